# Initial kernel scaffold; baseline (speedup 1.0000x reference)
import jax
import jax.numpy as jnp
from jax.experimental import pallas as pl


def kernel(x, router_w, w1, v1, w2):
    raise NotImplementedError("write your pallas kernel here")


# speedup vs baseline: 2.3674x; 2.3674x over previous
import functools

import jax
import jax.numpy as jnp
from jax import lax
from jax.experimental import pallas as pl
from jax.experimental.pallas import tpu as pltpu
from jax.experimental.pallas import tpu_sc as plsc

E = 8
K = 2
TILE = 512


def _dispatch_body(x_ref, rw_ref, d0_ref, d1_ref, g0_ref, g1_ref, te_ref,
                   nv_ref, nrows):
    s = x_ref.shape[0]
    ntile_pad = te_ref.shape[0]
    xf = x_ref[...]
    logits = jax.lax.dot_general(
        xf, rw_ref[...], (((1,), (1,)), ((), ())),
        preferred_element_type=jnp.float32)
    m = jnp.max(logits, axis=-1, keepdims=True)
    ex = jnp.exp(logits - m)
    w = ex / jnp.sum(ex, axis=-1, keepdims=True)

    iota_e = lax.broadcasted_iota(jnp.int32, (s, E), 1)
    big = jnp.int32(E)
    w1v = jnp.max(w, axis=-1, keepdims=True)
    e1 = jnp.min(jnp.where(w == w1v, iota_e, big), axis=-1, keepdims=True)
    w_m = jnp.where(iota_e == e1, -1.0, w)
    w2v = jnp.max(w_m, axis=-1, keepdims=True)
    e2 = jnp.min(jnp.where(w_m == w2v, iota_e, big), axis=-1, keepdims=True)
    denom = w1v + w2v
    g0_ref[...] = jnp.broadcast_to(w1v / denom, (s, 128))
    g1_ref[...] = jnp.broadcast_to(w2v / denom, (s, 128))

    oh1 = (iota_e == e1).astype(jnp.bfloat16)
    oh2 = (iota_e == e2).astype(jnp.bfloat16)
    tri = (lax.broadcasted_iota(jnp.int32, (s, s), 0)
           > lax.broadcasted_iota(jnp.int32, (s, s), 1)).astype(jnp.bfloat16)
    excl1 = jax.lax.dot_general(tri, oh1, (((1,), (0,)), ((), ())),
                                preferred_element_type=jnp.float32)
    excl2 = jax.lax.dot_general(tri, oh2, (((1,), (0,)), ((), ())),
                                preferred_element_type=jnp.float32)
    tot1 = jnp.sum(oh1.astype(jnp.float32), axis=0, keepdims=True)
    tot2 = jnp.sum(oh2.astype(jnp.float32), axis=0, keepdims=True)
    cnt = (tot1 + tot2).astype(jnp.int32)
    pc = ((cnt + (TILE - 1)) // TILE) * TILE
    tri_e = (lax.broadcasted_iota(jnp.int32, (E, E), 0)
             <= lax.broadcasted_iota(jnp.int32, (E, E), 1)).astype(jnp.float32)
    cum_pc = jax.lax.dot_general(
        pc.astype(jnp.float32), tri_e, (((1,), (0,)), ((), ())),
        preferred_element_type=jnp.float32).astype(jnp.int32)
    off = (cum_pc - pc).astype(jnp.float32)

    oh1f = oh1.astype(jnp.float32)
    oh2f = oh2.astype(jnp.float32)
    d0 = jnp.sum(oh1f * (off + excl1), axis=-1, keepdims=True)
    d1 = jnp.sum(oh2f * (off + tot1 + excl2), axis=-1, keepdims=True)
    d0_ref[...] = d0.astype(jnp.int32)
    d1_ref[...] = d1.astype(jnp.int32)

    total = jnp.sum(pc, axis=-1, keepdims=True)
    nvalid = total // TILE
    nv_ref[...] = nvalid
    rows = lax.broadcasted_iota(jnp.int32, (ntile_pad, E), 0) * TILE
    te_raw = jnp.sum((rows >= cum_pc).astype(jnp.int32), axis=-1,
                     keepdims=True)
    te_last = jnp.sum(((total - TILE) >= cum_pc).astype(jnp.int32), axis=-1,
                      keepdims=True)
    te_ref[...] = jnp.minimum(te_raw, te_last)
    del nrows


def _dispatch(xf, router_w, nrows, ntiles):
    s, _ = xf.shape
    ntile_pad = max(8, ntiles)
    outs = pl.pallas_call(
        functools.partial(_dispatch_body, nrows=nrows),
        out_shape=(
            jax.ShapeDtypeStruct((s, 1), jnp.int32),
            jax.ShapeDtypeStruct((s, 1), jnp.int32),
            jax.ShapeDtypeStruct((s, 128), jnp.float32),
            jax.ShapeDtypeStruct((s, 128), jnp.float32),
            jax.ShapeDtypeStruct((ntile_pad, 1), jnp.int32),
            jax.ShapeDtypeStruct((1, 1), jnp.int32),
        ),
    )(xf, router_w)
    return outs


def _make_scatter(s, h, nrows, nw):
    tok_w = s // nw
    mesh = plsc.VectorSubcoreMesh(core_axis_name="c", subcore_axis_name="s")

    @functools.partial(
        pl.kernel,
        out_type=(
            jax.ShapeDtypeStruct((nrows, h), jnp.float32),
            jax.ShapeDtypeStruct((nrows, 128), jnp.float32),
        ),
        mesh=mesh,
        scratch_types=[
            pltpu.VMEM((tok_w,), jnp.int32),
            pltpu.VMEM((tok_w,), jnp.int32),
            pltpu.VMEM((tok_w, h), jnp.float32),
            pltpu.VMEM((tok_w, 128), jnp.float32),
            pltpu.VMEM((tok_w, 128), jnp.float32),
            pltpu.SemaphoreType.DMA,
            pltpu.SemaphoreType.DMA,
            pltpu.SemaphoreType.DMA,
            pltpu.SemaphoreType.DMA,
        ],
    )
    def scatter_k(x_hbm, d0_hbm, d1_hbm, g0_hbm, g1_hbm,
                  xbuf_hbm, grow_hbm,
                  i0_v, i1_v, rows_v, ga_v, gb_v, sem0, sem1, sem2, sem3):
        wid = lax.axis_index("s") * 2 + lax.axis_index("c")
        base = wid * tok_w
        lx = pltpu.async_copy(x_hbm.at[pl.ds(base, tok_w)], rows_v, sem0)
        l0 = pltpu.async_copy(d0_hbm.at[pl.ds(base, tok_w)], i0_v, sem1)
        l1 = pltpu.async_copy(d1_hbm.at[pl.ds(base, tok_w)], i1_v, sem1)
        lg0 = pltpu.async_copy(g0_hbm.at[pl.ds(base, tok_w)], ga_v, sem2)
        lg1 = pltpu.async_copy(g1_hbm.at[pl.ds(base, tok_w)], gb_v, sem2)
        l0.wait()
        l1.wait()
        lx.wait()
        c0 = pltpu.async_copy(rows_v, xbuf_hbm.at[i0_v], sem0)
        c1 = pltpu.async_copy(rows_v, xbuf_hbm.at[i1_v], sem1)
        lg0.wait()
        lg1.wait()
        c2 = pltpu.async_copy(ga_v, grow_hbm.at[i0_v], sem2)
        c3 = pltpu.async_copy(gb_v, grow_hbm.at[i1_v], sem3)
        c0.wait()
        c1.wait()
        c2.wait()
        c3.wait()

    return scatter_k


def _ffn_body(te_ref, nv_ref, x_ref, w1_ref, v1_ref, w2_ref, g_ref, y_ref):
    i = pl.program_id(0)

    @pl.when(i < nv_ref[0])
    def _():
        xt = x_ref[...]
        a = jax.lax.dot_general(xt, w1_ref[0], (((1,), (1,)), ((), ())),
                                preferred_element_type=jnp.float32)
        b = jax.lax.dot_general(xt, v1_ref[0], (((1,), (1,)), ((), ())),
                                preferred_element_type=jnp.float32)
        hgl = (a * jax.nn.sigmoid(a)) * b
        y = jax.lax.dot_general(
            hgl, w2_ref[0], (((1,), (0,)), ((), ())),
            preferred_element_type=jnp.float32)
        y_ref[...] = y * g_ref[:, 0:1]


def _ffn(xbuf, w1r, v1r, w2r, grow, te, nv, nrows, ntiles):
    h = xbuf.shape[1]
    f = w1r.shape[1]
    wspec = pl.BlockSpec((1, f, h), lambda i, te_s, nv_s: (te_s[i], 0, 0))

    def live(i, nv_s):
        return jnp.minimum(i, nv_s[0] - 1)

    grid_spec = pltpu.PrefetchScalarGridSpec(
        num_scalar_prefetch=2,
        grid=(ntiles,),
        in_specs=[
            pl.BlockSpec((TILE, h), lambda i, te_s, nv_s: (live(i, nv_s), 0)),
            wspec, wspec, wspec,
            pl.BlockSpec((TILE, 128),
                         lambda i, te_s, nv_s: (live(i, nv_s), 0)),
        ],
        out_specs=pl.BlockSpec((TILE, h),
                               lambda i, te_s, nv_s: (live(i, nv_s), 0)),
    )
    return pl.pallas_call(
        _ffn_body,
        grid_spec=grid_spec,
        out_shape=jax.ShapeDtypeStruct((nrows, h), jnp.float32),
        compiler_params=pltpu.CompilerParams(
            dimension_semantics=("arbitrary",),
            vmem_limit_bytes=100 * 1024 * 1024),
    )(te, nv, xbuf, w1r, v1r, w2r, grow)


def _make_combine(s, h, nrows, nw, chunk):
    tok_w = s // nw
    nchunk = tok_w // chunk
    mesh = plsc.VectorSubcoreMesh(core_axis_name="c", subcore_axis_name="s")

    @functools.partial(
        pl.kernel,
        out_type=jax.ShapeDtypeStruct((s, h), jnp.float32),
        mesh=mesh,
        scratch_types=[
            pltpu.VMEM((tok_w,), jnp.int32),
            pltpu.VMEM((tok_w,), jnp.int32),
            pltpu.VMEM((2, chunk, h), jnp.float32),
            pltpu.VMEM((2, chunk, h), jnp.float32),
            pltpu.SemaphoreType.DMA,
            pltpu.SemaphoreType.DMA,
            pltpu.SemaphoreType.DMA,
        ],
    )
    def combine_k(y_hbm, d0_hbm, d1_hbm, out_hbm,
                  i0_v, i1_v, y0_v, y1_v, sem0, sem1, semw):
        wid = lax.axis_index("s") * 2 + lax.axis_index("c")
        base = wid * tok_w
        pltpu.sync_copy(d0_hbm.at[pl.ds(base, tok_w)], i0_v)
        pltpu.sync_copy(d1_hbm.at[pl.ds(base, tok_w)], i1_v)

        def start(c):
            iv0 = i0_v[pl.ds(c * chunk, chunk)]
            iv1 = i1_v[pl.ds(c * chunk, chunk)]
            a = pltpu.async_copy(y_hbm.at[iv0], y0_v.at[c % 2], sem0)
            b = pltpu.async_copy(y_hbm.at[iv1], y1_v.at[c % 2], sem1)
            return a, b

        cp = start(0)
        prev_w = None
        for c in range(nchunk):
            cp[0].wait()
            cp[1].wait()
            if prev_w is not None:
                prev_w.wait()
            if c + 1 < nchunk:
                cp = start(c + 1)
            buf = c % 2

            def row_body(r, _):
                for j in range(h // 16):
                    sl = pl.ds(j * 16, 16)
                    y0_v[buf, r, sl] = y0_v[buf, r, sl] + y1_v[buf, r, sl]
                return 0

            lax.fori_loop(0, chunk, row_body, 0)
            prev_w = pltpu.async_copy(
                y0_v.at[buf], out_hbm.at[pl.ds(base + c * chunk, chunk)],
                semw)
        prev_w.wait()

    return combine_k


def kernel(x, router_w, w1, v1, w2):
    b, s, h = x.shape
    e = router_w.shape[0]
    f = w1.shape[0] // e
    nrows = s * K + e * TILE
    ntiles = nrows // TILE
    nw = 32

    xf = x.reshape(s, h).astype(jnp.float32)
    d0, d1, g0, g1, te, nv = _dispatch(xf, router_w, nrows, ntiles)
    d0f = d0.reshape(s)
    d1f = d1.reshape(s)

    xbuf, grow = _make_scatter(s, h, nrows, nw)(xf, d0f, d1f, g0, g1)
    y = _ffn(xbuf, w1.reshape(e, f, h), v1.reshape(e, f, h),
             w2.reshape(e, f, h), grow, te[:ntiles, 0], nv.reshape(1),
             nrows, ntiles)
    out = _make_combine(s, h, nrows, nw, 16)(y, d0f, d1f)
    return out.reshape(b, s, h).astype(x.dtype)
